# baseline (speedup 1.0000x reference)
import functools

import jax
import jax.numpy as jnp
from jax import lax
from jax.experimental import pallas as pl
from jax.experimental.pallas import tpu as pltpu
from jax.experimental.pallas import tpu_sc as plsc

N = 10000
D = 128
E = 320000
NC, NS = 2, 16
NW = NC * NS
NP = 10240
ROWS_PER_TILE = NP // NS
EW = E // NW
EPAD = 327680
EROWS = EPAD // 128
WROWS = EROWS // NW
EWP = EPAD // NW
B = 128
ITERS = EWP // B
BP = 80
ITERS_P = EWP // BP

_mesh = plsc.VectorSubcoreMesh(
    core_axis_name="c", subcore_axis_name="s", num_cores=NC, num_subcores=NS
)


def _deg_body(ei_hbm, out_hbm, slab, dstslab, onesv, zv, obuf, accum, sem):
    c = lax.axis_index("c")
    s = lax.axis_index("s")
    wid = c * NS + s
    zeros16 = jnp.zeros((16,), jnp.float32)
    ones16 = jnp.ones((16,), jnp.float32)

    def zfill(i, _):
        zv[pl.ds(i * 16, 16)] = zeros16
        return 0

    lax.fori_loop(0, ROWS_PER_TILE // 16, zfill, 0)
    for j in range(B // 16):
        onesv[pl.ds(j * 16, 16)] = ones16
    pltpu.sync_copy(ei_hbm.at[pl.ds(wid * WROWS, WROWS)], slab)
    pltpu.sync_copy(zv, accum.at[pl.ds(s * ROWS_PER_TILE, ROWS_PER_TILE)])

    def dfill(i, _):
        for j in range(B // 16):
            v = slab[i, pl.ds(16 * j, 16)]
            dstslab[i, pl.ds(16 * j, 16)] = jnp.right_shift(v, 16)
        return 0

    lax.fori_loop(0, ITERS, dfill, 0)
    plsc.subcore_barrier()

    def body(i, _):
        pltpu.async_copy(onesv, accum.at[dstslab.at[i]], sem, add=True)
        return 0

    lax.fori_loop(0, ITERS, body, 0)

    def drain(i, _):
        pltpu.make_async_copy(onesv, accum.at[dstslab.at[0]], sem).wait()
        return 0

    lax.fori_loop(0, ITERS, drain, 0)
    plsc.subcore_barrier()
    pltpu.sync_copy(accum.at[pl.ds(s * ROWS_PER_TILE, ROWS_PER_TILE)], obuf)
    pltpu.sync_copy(obuf, out_hbm.at[c, pl.ds(s * ROWS_PER_TILE, ROWS_PER_TILE)])


_deg_call = pl.kernel(
    _deg_body,
    jax.ShapeDtypeStruct((NC, NP), jnp.float32),
    mesh=_mesh,
    scratch_types=[
        pltpu.VMEM((WROWS, 128), jnp.int32),
        pltpu.VMEM((WROWS, 128), jnp.int32),
        pltpu.VMEM((B,), jnp.float32),
        pltpu.VMEM((ROWS_PER_TILE,), jnp.float32),
        pltpu.VMEM((ROWS_PER_TILE,), jnp.float32),
        pltpu.VMEM_SHARED((NP,), jnp.float32),
        pltpu.SemaphoreType.DMA,
    ],
    name="sc_gcn_deg",
)


def _prop_body(g_hbm, ei_hbm, out_hbm, slab, wsrc_a, wsrc_b, wsrc_c, wdst_a,
               wdst_b, wdst_c, rows_a, rows_b, rows_c, accum,
               gsem_a, gsem_b, gsem_c, ssem_a, ssem_b, ssem_c):
    c = lax.axis_index("c")
    s = lax.axis_index("s")
    wid = c * NS + s
    zeros16 = jnp.zeros((16,), jnp.float32)

    def zfill(i, _):
        for j in range(D // 16):
            rows_a[i, pl.ds(j * 16, 16)] = zeros16
        return 0

    lax.fori_loop(0, BP, zfill, 0)
    pltpu.sync_copy(ei_hbm.at[pl.ds(wid * WROWS, WROWS)], slab)

    def zcopy(r, _):
        pltpu.sync_copy(rows_a, accum.at[pl.ds(s * ROWS_PER_TILE + r * BP, BP)])
        return 0

    lax.fori_loop(0, ROWS_PER_TILE // BP, zcopy, 0)
    plsc.subcore_barrier()

    def unpack(i, wsrc, wdst):
        for j in range(BP // 16):
            o16 = i * (BP // 16) + j
            v = slab[jnp.right_shift(o16, 3),
                     pl.ds(jnp.left_shift(jnp.bitwise_and(o16, 7), 4), 16)]
            wsrc[pl.ds(16 * j, 16)] = jnp.bitwise_and(v, 0xFFFF)
            wdst[pl.ds(16 * j, 16)] = jnp.right_shift(v, 16)

    def g_start(widx, buf, sem):
        pltpu.async_copy(g_hbm.at[widx], buf, sem)

    def g_wait(buf, sem):
        pltpu.make_async_copy(g_hbm.at[wsrc_a], buf, sem).wait()

    def s_start(widx, buf, sem):
        pltpu.async_copy(buf, accum.at[widx], sem, add=True)

    def s_wait(buf, sem):
        pltpu.make_async_copy(buf, accum.at[wdst_a], sem).wait()

    bufs = [(wsrc_a, wdst_a, rows_a, gsem_a, ssem_a),
            (wsrc_b, wdst_b, rows_b, gsem_b, ssem_b),
            (wsrc_c, wdst_c, rows_c, gsem_c, ssem_c)]

    def chunk(i, x, z, first=False, lookahead=True):
        xs, xd, xr, xg, xss = bufs[x]
        zs, zd, zr, zg, zss = bufs[z]
        g_wait(xr, xg)
        s_start(xd, xr, xss)
        if not first:
            s_wait(zr, zss)
        if lookahead:
            unpack(i + 2, zs, zd)
            g_start(zs, zr, zg)

    unpack(0, wsrc_a, wdst_a)
    g_start(wsrc_a, rows_a, gsem_a)
    unpack(1, wsrc_b, wdst_b)
    g_start(wsrc_b, rows_b, gsem_b)
    chunk(0, 0, 2, first=True)

    def body(k, _):
        i1 = 3 * k + 1
        chunk(i1, 1, 0)
        chunk(i1 + 1, 2, 1)
        chunk(i1 + 2, 0, 2)
        return 0

    lax.fori_loop(0, (ITERS_P - 5) // 3, body, 0)
    chunk(ITERS_P - 4, 1, 0)
    chunk(ITERS_P - 3, 2, 1)
    chunk(ITERS_P - 2, 0, 2, lookahead=False)
    chunk(ITERS_P - 1, 1, 0, lookahead=False)
    s_wait(rows_b, ssem_b)

    plsc.subcore_barrier()

    def wb(r, _):
        sl = pl.ds(s * ROWS_PER_TILE + r * BP, BP)
        pltpu.sync_copy(accum.at[sl], rows_a)
        pltpu.sync_copy(rows_a, out_hbm.at[c, sl])
        return 0

    lax.fori_loop(0, ROWS_PER_TILE // BP, wb, 0)


_prop_call = pl.kernel(
    _prop_body,
    jax.ShapeDtypeStruct((NC, NP, D), jnp.float32),
    mesh=_mesh,
    scratch_types=[
        pltpu.VMEM((WROWS, 128), jnp.int32),
        pltpu.VMEM((BP,), jnp.int32),
        pltpu.VMEM((BP,), jnp.int32),
        pltpu.VMEM((BP,), jnp.int32),
        pltpu.VMEM((BP,), jnp.int32),
        pltpu.VMEM((BP,), jnp.int32),
        pltpu.VMEM((BP,), jnp.int32),
        pltpu.VMEM((BP, D), jnp.float32),
        pltpu.VMEM((BP, D), jnp.float32),
        pltpu.VMEM((BP, D), jnp.float32),
        pltpu.VMEM_SHARED((NP, D), jnp.float32),
        pltpu.SemaphoreType.DMA,
        pltpu.SemaphoreType.DMA,
        pltpu.SemaphoreType.DMA,
        pltpu.SemaphoreType.DMA,
        pltpu.SemaphoreType.DMA,
        pltpu.SemaphoreType.DMA,
    ],
    name="sc_gcn_prop",
)


BN = 2000


def _pre_body(d2, x, w, dinv, g1):
    dd = d2[...]
    deg = dd[:, 0:1] + dd[:, 1:2] + 1.0
    di = lax.rsqrt(deg)
    dinv[...] = di
    h1 = jnp.dot(x[...], w[...], preferred_element_type=jnp.float32)
    g1[...] = h1 * di


def _mid_body(s2, g1, dinv, w, b, g2):
    di = dinv[...]
    r = jnp.maximum((s2[0] + s2[1] + g1[...]) * di + b[...], 0.0)
    h2 = jnp.dot(r, w[...], preferred_element_type=jnp.float32)
    g2[...] = h2 * di


def _fin_body(t2, g2, dinv, b, out):
    di = dinv[...]
    out[...] = (t2[0] + t2[1] + g2[...]) * di + b[...]


_pre_call = pl.pallas_call(
    _pre_body,
    grid=(N // BN,),
    in_specs=[
        pl.BlockSpec((BN, NC), lambda i: (i, 0)),
        pl.BlockSpec((BN, D), lambda i: (i, 0)),
        pl.BlockSpec((D, D), lambda i: (0, 0)),
    ],
    out_specs=[
        pl.BlockSpec((BN, 1), lambda i: (i, 0)),
        pl.BlockSpec((BN, D), lambda i: (i, 0)),
    ],
    out_shape=[
        jax.ShapeDtypeStruct((N, 1), jnp.float32),
        jax.ShapeDtypeStruct((N, D), jnp.float32),
    ],
    name="tc_gcn_pre",
)

_mid_call = pl.pallas_call(
    _mid_body,
    grid=(N // BN,),
    in_specs=[
        pl.BlockSpec((NC, BN, D), lambda i: (0, i, 0)),
        pl.BlockSpec((BN, D), lambda i: (i, 0)),
        pl.BlockSpec((BN, 1), lambda i: (i, 0)),
        pl.BlockSpec((D, D), lambda i: (0, 0)),
        pl.BlockSpec((1, D), lambda i: (0, 0)),
    ],
    out_specs=pl.BlockSpec((BN, D), lambda i: (i, 0)),
    out_shape=jax.ShapeDtypeStruct((N, D), jnp.float32),
    name="tc_gcn_mid",
)

_fin_call = pl.pallas_call(
    _fin_body,
    grid=(N // BN,),
    in_specs=[
        pl.BlockSpec((NC, BN, D), lambda i: (0, i, 0)),
        pl.BlockSpec((BN, D), lambda i: (i, 0)),
        pl.BlockSpec((BN, 1), lambda i: (i, 0)),
        pl.BlockSpec((1, D), lambda i: (0, 0)),
    ],
    out_specs=pl.BlockSpec((BN, D), lambda i: (i, 0)),
    out_shape=jax.ShapeDtypeStruct((N, D), jnp.float32),
    name="tc_gcn_fin",
)


@jax.jit
def kernel(x, edge_index, W1, b1, W2, b2):
    ei32 = edge_index.astype(jnp.int32)
    packed = jnp.bitwise_or(ei32[0], jnp.left_shift(ei32[1], 16))
    pdst = jnp.broadcast_to(
        jnp.arange(NP - N, dtype=jnp.int32), ((EPAD - E) // (NP - N), NP - N)
    ).reshape(EPAD - E) + N
    psrc = jnp.arange(EPAD - E, dtype=jnp.int32)
    ei = jnp.concatenate(
        [packed, jnp.bitwise_or(psrc, jnp.left_shift(pdst, 16))]
    ).reshape(EROWS, 128)
    degp = _deg_call(ei)
    degt = degp.T
    dinv, g1 = _pre_call(degt, x, W1)
    s = _prop_call(g1, ei)
    g2 = _mid_call(s, g1, dinv, W2, b1.reshape(1, D))
    t = _prop_call(g2, ei)
    out = _fin_call(t, g2, dinv, b2.reshape(1, D))
    return out


# speedup vs baseline: 1.0053x; 1.0053x over previous
import functools

import jax
import jax.numpy as jnp
from jax import lax
from jax.experimental import pallas as pl
from jax.experimental.pallas import tpu as pltpu
from jax.experimental.pallas import tpu_sc as plsc

N = 10000
D = 128
E = 320000
NC, NS = 2, 16
NW = NC * NS
NP = 10240
ROWS_PER_TILE = NP // NS
EW = E // NW
B = 80
ITERS = EW // B
BP = 80
ITERS_P = EW // BP

_mesh = plsc.VectorSubcoreMesh(
    core_axis_name="c", subcore_axis_name="s", num_cores=NC, num_subcores=NS
)


def _deg_body(dst_hbm, out_hbm, idxd, onesv, zv, obuf, accum, sem):
    c = lax.axis_index("c")
    s = lax.axis_index("s")
    wid = c * NS + s
    zeros16 = jnp.zeros((16,), jnp.float32)
    ones16 = jnp.ones((16,), jnp.float32)

    def zfill(i, _):
        zv[pl.ds(i * 16, 16)] = zeros16
        return 0

    lax.fori_loop(0, ROWS_PER_TILE // 16, zfill, 0)
    for j in range(B // 16):
        onesv[pl.ds(j * 16, 16)] = ones16
    pltpu.sync_copy(dst_hbm.at[1, wid], idxd)
    pltpu.sync_copy(zv, accum.at[pl.ds(s * ROWS_PER_TILE, ROWS_PER_TILE)])
    plsc.subcore_barrier()

    def body(i, _):
        pltpu.async_copy(onesv, accum.at[idxd.at[i]], sem, add=True)
        return 0

    lax.fori_loop(0, ITERS, body, 0)

    def drain(i, _):
        pltpu.make_async_copy(onesv, accum.at[idxd.at[0]], sem).wait()
        return 0

    lax.fori_loop(0, ITERS, drain, 0)
    plsc.subcore_barrier()
    pltpu.sync_copy(accum.at[pl.ds(s * ROWS_PER_TILE, ROWS_PER_TILE)], obuf)
    pltpu.sync_copy(obuf, out_hbm.at[c, pl.ds(s * ROWS_PER_TILE, ROWS_PER_TILE)])


_deg_call = pl.kernel(
    _deg_body,
    jax.ShapeDtypeStruct((NC, NP), jnp.float32),
    mesh=_mesh,
    scratch_types=[
        pltpu.VMEM((ITERS, B), jnp.int32),
        pltpu.VMEM((B,), jnp.float32),
        pltpu.VMEM((ROWS_PER_TILE,), jnp.float32),
        pltpu.VMEM((ROWS_PER_TILE,), jnp.float32),
        pltpu.VMEM_SHARED((NP,), jnp.float32),
        pltpu.SemaphoreType.DMA,
    ],
    name="sc_gcn_deg",
)


def _prop_body(g_hbm, ei_hbm, out_hbm, slab, wsrc_a, wsrc_b, wsrc_c, wdst_a,
               wdst_b, wdst_c, rows_a, rows_b, rows_c, accum,
               gsem_a, gsem_b, gsem_c, ssem_a, ssem_b, ssem_c):
    c = lax.axis_index("c")
    s = lax.axis_index("s")
    wid = c * NS + s
    zeros16 = jnp.zeros((16,), jnp.float32)

    def zfill(i, _):
        for j in range(D // 16):
            rows_a[i, pl.ds(j * 16, 16)] = zeros16
        return 0

    lax.fori_loop(0, BP, zfill, 0)
    pltpu.sync_copy(ei_hbm.at[wid], slab)

    def zcopy(r, _):
        pltpu.sync_copy(rows_a, accum.at[pl.ds(s * ROWS_PER_TILE + r * BP, BP)])
        return 0

    lax.fori_loop(0, ROWS_PER_TILE // BP, zcopy, 0)
    plsc.subcore_barrier()

    def unpack(i, wsrc, wdst):
        for j in range(BP // 16):
            v = slab[i, pl.ds(16 * j, 16)]
            wsrc[pl.ds(16 * j, 16)] = jnp.bitwise_and(v, 0xFFFF)
            wdst[pl.ds(16 * j, 16)] = jnp.right_shift(v, 16)

    def g_start(widx, buf, sem):
        pltpu.async_copy(g_hbm.at[widx], buf, sem)

    def g_wait(buf, sem):
        pltpu.make_async_copy(g_hbm.at[wsrc_a], buf, sem).wait()

    def s_start(widx, buf, sem):
        pltpu.async_copy(buf, accum.at[widx], sem, add=True)

    def s_wait(buf, sem):
        pltpu.make_async_copy(buf, accum.at[wdst_a], sem).wait()

    bufs = [(wsrc_a, wdst_a, rows_a, gsem_a, ssem_a),
            (wsrc_b, wdst_b, rows_b, gsem_b, ssem_b),
            (wsrc_c, wdst_c, rows_c, gsem_c, ssem_c)]

    def chunk(i, x, z, first=False, lookahead=True):
        xs, xd, xr, xg, xss = bufs[x]
        zs, zd, zr, zg, zss = bufs[z]
        g_wait(xr, xg)
        s_start(xd, xr, xss)
        if not first:
            s_wait(zr, zss)
        if lookahead:
            unpack(i + 2, zs, zd)
            g_start(zs, zr, zg)

    unpack(0, wsrc_a, wdst_a)
    g_start(wsrc_a, rows_a, gsem_a)
    unpack(1, wsrc_b, wdst_b)
    g_start(wsrc_b, rows_b, gsem_b)
    chunk(0, 0, 2, first=True)

    def body(k, _):
        i1 = 3 * k + 1
        chunk(i1, 1, 0)
        chunk(i1 + 1, 2, 1)
        chunk(i1 + 2, 0, 2)
        return 0

    lax.fori_loop(0, (ITERS_P - 5) // 3, body, 0)
    chunk(ITERS_P - 4, 1, 0)
    chunk(ITERS_P - 3, 2, 1)
    chunk(ITERS_P - 2, 0, 2, lookahead=False)
    chunk(ITERS_P - 1, 1, 0, lookahead=False)
    s_wait(rows_b, ssem_b)

    plsc.subcore_barrier()

    def wb(r, _):
        sl = pl.ds(s * ROWS_PER_TILE + r * BP, BP)
        pltpu.sync_copy(accum.at[sl], rows_a)
        pltpu.sync_copy(rows_a, out_hbm.at[c, sl])
        return 0

    lax.fori_loop(0, ROWS_PER_TILE // BP, wb, 0)


_prop_call = pl.kernel(
    _prop_body,
    jax.ShapeDtypeStruct((NC, NP, D), jnp.float32),
    mesh=_mesh,
    scratch_types=[
        pltpu.VMEM((ITERS_P, BP), jnp.int32),
        pltpu.VMEM((BP,), jnp.int32),
        pltpu.VMEM((BP,), jnp.int32),
        pltpu.VMEM((BP,), jnp.int32),
        pltpu.VMEM((BP,), jnp.int32),
        pltpu.VMEM((BP,), jnp.int32),
        pltpu.VMEM((BP,), jnp.int32),
        pltpu.VMEM((BP, D), jnp.float32),
        pltpu.VMEM((BP, D), jnp.float32),
        pltpu.VMEM((BP, D), jnp.float32),
        pltpu.VMEM_SHARED((NP, D), jnp.float32),
        pltpu.SemaphoreType.DMA,
        pltpu.SemaphoreType.DMA,
        pltpu.SemaphoreType.DMA,
        pltpu.SemaphoreType.DMA,
        pltpu.SemaphoreType.DMA,
        pltpu.SemaphoreType.DMA,
    ],
    name="sc_gcn_prop",
)


BN = 2000


def _pre_body(d2, x, w, dinv, g1):
    dd = d2[...]
    deg = dd[:, 0:1] + dd[:, 1:2] + 1.0
    di = lax.rsqrt(deg)
    dinv[...] = di
    h1 = jnp.dot(x[...], w[...], preferred_element_type=jnp.float32)
    g1[...] = h1 * di


def _mid_body(s2, g1, dinv, w, b, g2):
    di = dinv[...]
    r = jnp.maximum((s2[0] + s2[1] + g1[...]) * di + b[...], 0.0)
    h2 = jnp.dot(r, w[...], preferred_element_type=jnp.float32)
    g2[...] = h2 * di


def _fin_body(t2, g2, dinv, b, out):
    di = dinv[...]
    out[...] = (t2[0] + t2[1] + g2[...]) * di + b[...]


_pre_call = pl.pallas_call(
    _pre_body,
    grid=(N // BN,),
    in_specs=[
        pl.BlockSpec((BN, NC), lambda i: (i, 0)),
        pl.BlockSpec((BN, D), lambda i: (i, 0)),
        pl.BlockSpec((D, D), lambda i: (0, 0)),
    ],
    out_specs=[
        pl.BlockSpec((BN, 1), lambda i: (i, 0)),
        pl.BlockSpec((BN, D), lambda i: (i, 0)),
    ],
    out_shape=[
        jax.ShapeDtypeStruct((N, 1), jnp.float32),
        jax.ShapeDtypeStruct((N, D), jnp.float32),
    ],
    name="tc_gcn_pre",
)

_mid_call = pl.pallas_call(
    _mid_body,
    grid=(N // BN,),
    in_specs=[
        pl.BlockSpec((NC, BN, D), lambda i: (0, i, 0)),
        pl.BlockSpec((BN, D), lambda i: (i, 0)),
        pl.BlockSpec((BN, 1), lambda i: (i, 0)),
        pl.BlockSpec((D, D), lambda i: (0, 0)),
        pl.BlockSpec((1, D), lambda i: (0, 0)),
    ],
    out_specs=pl.BlockSpec((BN, D), lambda i: (i, 0)),
    out_shape=jax.ShapeDtypeStruct((N, D), jnp.float32),
    name="tc_gcn_mid",
)

_fin_call = pl.pallas_call(
    _fin_body,
    grid=(N // BN,),
    in_specs=[
        pl.BlockSpec((NC, BN, D), lambda i: (0, i, 0)),
        pl.BlockSpec((BN, D), lambda i: (i, 0)),
        pl.BlockSpec((BN, 1), lambda i: (i, 0)),
        pl.BlockSpec((1, D), lambda i: (0, 0)),
    ],
    out_specs=pl.BlockSpec((BN, D), lambda i: (i, 0)),
    out_shape=jax.ShapeDtypeStruct((N, D), jnp.float32),
    name="tc_gcn_fin",
)


@jax.jit
def kernel(x, edge_index, W1, b1, W2, b2):
    ei32 = edge_index.astype(jnp.int32)
    degp = _deg_call(ei32.reshape(2, NW, ITERS, B))
    degt = degp.T
    srcv = ei32[0]
    dstv = ei32[1]
    packed = jnp.bitwise_or(srcv, jnp.left_shift(dstv, 16))
    ei = packed.reshape(NW, ITERS_P, BP)
    dinv, g1 = _pre_call(degt, x, W1)
    s = _prop_call(g1, ei)
    g2 = _mid_call(s, g1, dinv, W2, b1.reshape(1, D))
    t = _prop_call(g2, ei)
    out = _fin_call(t, g2, dinv, b2.reshape(1, D))
    return out
